# Initial kernel scaffold; baseline (speedup 1.0000x reference)
import jax
import jax.numpy as jnp
from jax.experimental import pallas as pl


def kernel(x, edge_index, edge_attr, params):
    raise NotImplementedError("write your pallas kernel here")


# speedup vs baseline: 2.4925x; 2.4925x over previous
import functools

import jax
import jax.numpy as jnp
from jax import lax
from jax.experimental import pallas as pl
from jax.experimental.pallas import tpu as pltpu
from jax.experimental.pallas import tpu_sc as plsc

N_NODES = 10000
N_EDGES = 320000
HID = 128
EDGE_DIM = 16

NC = 2
NS = 16
L = 16
NW = NC * NS
EPT = N_EDGES // NW
CHUNK = 80
NCHUNK = EPT // CHUNK
N_PAD = 10240
ROWS_PT = N_PAD // NS
ZROWS = 128

_mesh = plsc.VectorSubcoreMesh(core_axis_name="c", subcore_axis_name="s")


@functools.partial(
    pl.kernel,
    out_type=jax.ShapeDtypeStruct((NC, N_PAD, HID), jnp.float32),
    mesh=_mesh,
    scratch_types=[
        pltpu.VMEM((CHUNK,), jnp.int32),
        pltpu.VMEM((CHUNK,), jnp.int32),
        pltpu.VMEM((CHUNK, HID), jnp.float32),
        pltpu.VMEM((CHUNK, HID), jnp.float32),
        pltpu.VMEM((ZROWS, HID), jnp.float32),
        pltpu.VMEM_SHARED((N_PAD, HID), jnp.float32),
        pltpu.SemaphoreType.DMA,
    ],
)
def _mp_sc(x_hbm, e_hbm, src_hbm, dst_hbm, out_hbm,
           src_v, dst_v, xr_v, e_v, z_v, aggr_s, sem):
    cid = lax.axis_index("c")
    sid = lax.axis_index("s")
    wid = cid * NS + sid

    def _zrow(i, carry):
        for b in range(HID // L):
            z_v[i, pl.ds(b * L, L)] = jnp.zeros((L,), jnp.float32)
        return carry

    lax.fori_loop(0, ZROWS, _zrow, 0)
    for j in range(ROWS_PT // ZROWS):
        pltpu.sync_copy(z_v, aggr_s.at[pl.ds(sid * ROWS_PT + j * ZROWS, ZROWS)])
    plsc.subcore_barrier()

    base = wid * EPT

    def _chunk(g, carry):
        off = base + g * CHUNK
        pltpu.sync_copy(src_hbm.at[pl.ds(off, CHUNK)], src_v)
        pltpu.sync_copy(dst_hbm.at[pl.ds(off, CHUNK)], dst_v)
        pltpu.async_copy(x_hbm.at[src_v], xr_v, sem).wait()
        pltpu.sync_copy(e_hbm.at[pl.ds(off, CHUNK)], e_v)

        def _row(i, c2):
            for b in range(HID // L):
                s = pl.ds(b * L, L)
                e_v[i, s] = jnp.maximum(xr_v[i, s] + e_v[i, s], 0.0)
            return c2

        lax.fori_loop(0, CHUNK, _row, 0)
        pltpu.sync_copy(e_v, aggr_s.at[dst_v], add=True)
        return carry

    lax.fori_loop(0, NCHUNK, _chunk, 0)
    plsc.subcore_barrier()
    pltpu.sync_copy(aggr_s.at[pl.ds(sid * ROWS_PT, ROWS_PT)],
                    out_hbm.at[cid, pl.ds(sid * ROWS_PT, ROWS_PT)])


def _edge_mlp_body(ea_ref, w0, b0, w1, b1, w2, b2, o0, o1, o2):
    a = ea_ref[...]
    o0[...] = jnp.dot(a, w0[...], preferred_element_type=jnp.float32) + b0[...]
    o1[...] = jnp.dot(a, w1[...], preferred_element_type=jnp.float32) + b1[...]
    o2[...] = jnp.dot(a, w2[...], preferred_element_type=jnp.float32) + b2[...]


_EBLK = 8000


def _edge_mlp(edge_attr, wts, bs):
    grid = (N_EDGES // _EBLK,)
    espec = pl.BlockSpec((_EBLK, EDGE_DIM), lambda i: (i, 0))
    wspec = pl.BlockSpec((EDGE_DIM, HID), lambda i: (0, 0))
    bspec = pl.BlockSpec((1, HID), lambda i: (0, 0))
    ospec = pl.BlockSpec((_EBLK, HID), lambda i: (i, 0))
    oshape = jax.ShapeDtypeStruct((N_EDGES, HID), jnp.float32)
    return pl.pallas_call(
        _edge_mlp_body,
        grid=grid,
        in_specs=[espec, wspec, bspec, wspec, bspec, wspec, bspec],
        out_specs=[ospec, ospec, ospec],
        out_shape=[oshape, oshape, oshape],
    )(edge_attr, wts[0], bs[0], wts[1], bs[1], wts[2], bs[2])


def _node_body(h_ref, p_ref, s_ref, w1_ref, b1_ref, g1_ref, bt1_ref,
               w2_ref, b2_ref, gn_ref, bn_ref, o_ref, *, residual):
    h = h_ref[...]
    a = s_ref[0, 0] * h + p_ref[0, :N_NODES] + p_ref[1, :N_NODES]
    t = jnp.dot(a, w1_ref[...], preferred_element_type=jnp.float32) + b1_ref[...]
    m = jnp.mean(t, axis=0, keepdims=True)
    v = jnp.mean((t - m) * (t - m), axis=0, keepdims=True)
    t = g1_ref[...] * (t - m) * lax.rsqrt(v + 1e-5) + bt1_ref[...]
    t = jnp.maximum(t, 0.0)
    u = jnp.dot(t, w2_ref[...], preferred_element_type=jnp.float32) + b2_ref[...]
    m2 = jnp.mean(u, axis=0, keepdims=True)
    v2 = jnp.mean((u - m2) * (u - m2), axis=0, keepdims=True)
    u = gn_ref[...] * (u - m2) * lax.rsqrt(v2 + 1e-5) + bn_ref[...]
    u = jnp.maximum(u, 0.0)
    o_ref[...] = h + 0.3 * u if residual else u


def _node_update(h, partials, p, residual):
    s = (1.0 + p['eps']).reshape(1, 1).astype(jnp.float32)
    body = functools.partial(_node_body, residual=residual)
    return pl.pallas_call(
        body,
        out_shape=jax.ShapeDtypeStruct((N_NODES, HID), jnp.float32),
    )(h, partials, s,
      p['W1'].T, p['b1'].reshape(1, -1), p['g1'].reshape(1, -1),
      p['bt1'].reshape(1, -1),
      p['W2'].T, p['b2'].reshape(1, -1), p['gn'].reshape(1, -1),
      p['bn'].reshape(1, -1))


def kernel(x, edge_index, edge_attr, params):
    src = edge_index[0].astype(jnp.int32)
    dst = edge_index[1].astype(jnp.int32)
    e_all = _edge_mlp(
        edge_attr,
        [params[i]['We'].T for i in range(3)],
        [params[i]['be'].reshape(1, -1) for i in range(3)],
    )
    h = x
    for i in range(3):
        partials = _mp_sc(h, e_all[i], src, dst)
        h = _node_update(h, partials, params[i], residual=(i == 1))
    return h


# speedup vs baseline: 4.0210x; 1.6132x over previous
import functools

import jax
import jax.numpy as jnp
import numpy as np
from jax import lax
from jax.experimental import pallas as pl
from jax.experimental.pallas import tpu as pltpu
from jax.experimental.pallas import tpu_sc as plsc

N_NODES = 10000
N_EDGES = 320000
HID = 128
EDGE_DIM = 16

NC = 2
NS = 16
L = 16
NW = NC * NS
EPT = N_EDGES // NW
CHUNK = 80
NCHUNK = EPT // CHUNK
N_PAD = 10240
ROWS_PT = N_PAD // NS

_SIGMA = np.concatenate(
    [32 * j + np.stack([np.arange(16), np.arange(16) + 16], axis=1).ravel()
     for j in range(4)])

_mesh = plsc.VectorSubcoreMesh(core_axis_name="c", subcore_axis_name="s")


@functools.partial(
    pl.kernel,
    out_type=jax.ShapeDtypeStruct((NC, N_PAD, HID), jnp.float32),
    mesh=_mesh,
    scratch_types=[
        pltpu.VMEM((2, CHUNK), jnp.int32),
        pltpu.VMEM((4, CHUNK), jnp.int32),
        pltpu.VMEM((2, CHUNK, HID), jnp.float32),
        pltpu.VMEM((2, CHUNK, HID), jnp.float32),
        pltpu.VMEM_SHARED((N_PAD, HID), jnp.float32),
    ] + [pltpu.SemaphoreType.DMA] * 8,
)
def _mp_sc(x_hbm, e_hbm, src_hbm, dst_hbm, out_hbm,
           src_v, dst_v, xr_v, e_v, aggr_s, *sems):
    cid = lax.axis_index("c")
    sid = lax.axis_index("s")
    wid = cid * NS + sid
    sem_i = sems[0:2]
    sem_e = sems[2:4]
    sem_g = sems[4:6]
    sem_s = sems[6:8]

    def _zrow(i, carry):
        for k in range(HID // L):
            xr_v[0, i, pl.ds(k * L, L)] = jnp.zeros((L,), jnp.float32)
        return carry

    lax.fori_loop(0, CHUNK, _zrow, 0)
    for j in range(ROWS_PT // CHUNK):
        pltpu.sync_copy(xr_v.at[0],
                        aggr_s.at[pl.ds(sid * ROWS_PT + j * CHUNK, CHUNK)])
    plsc.subcore_barrier()

    base = wid * EPT

    def _front(c, b, d4):
        off = base + c * CHUNK
        pltpu.async_copy(src_hbm.at[pl.ds(off, CHUNK)], src_v.at[b], sem_i[b])
        pltpu.async_copy(dst_hbm.at[pl.ds(off, CHUNK)], dst_v.at[d4], sem_i[b])
        pltpu.async_copy(e_hbm.at[pl.ds(off, CHUNK)], e_v.at[b], sem_e[b])

    def _wait_idx(c, b, d4):
        off = base + c * CHUNK
        pltpu.make_async_copy(src_hbm.at[pl.ds(off, CHUNK)], src_v.at[b],
                              sem_i[b]).wait()
        pltpu.make_async_copy(dst_hbm.at[pl.ds(off, CHUNK)], dst_v.at[d4],
                              sem_i[b]).wait()

    def _issue_gather(b):
        pltpu.async_copy(x_hbm.at[src_v.at[b]], xr_v.at[b], sem_g[b])

    def _wait_gather(b):
        pltpu.make_async_copy(x_hbm.at[src_v.at[b]], xr_v.at[b],
                              sem_g[b]).wait()

    def _wait_e(c, b):
        off = base + c * CHUNK
        pltpu.make_async_copy(e_hbm.at[pl.ds(off, CHUNK)], e_v.at[b],
                              sem_e[b]).wait()

    def _scatter(b, d4):
        pltpu.async_copy(xr_v.at[b], aggr_s.at[dst_v.at[d4]], sem_s[b],
                         add=True)

    def _wait_scatter(b, d4):
        pltpu.make_async_copy(xr_v.at[b], aggr_s.at[dst_v.at[d4]],
                              sem_s[b]).wait()

    def _compute(b):
        def _row(i, carry):
            for j in range(HID // L):
                s = pl.ds(j * L, L)
                xr_v[b, i, s] = jnp.maximum(xr_v[b, i, s] + e_v[b, i, s], 0.0)
            return carry

        lax.fori_loop(0, CHUNK, _row, 0)

    _front(0, 0, 0)
    _wait_idx(0, 0, 0)
    _issue_gather(0)
    _front(1, 1, 1)

    @pl.loop(0, NCHUNK - 1, step=4)
    def _grp(g):
        for u in range(4):
            c = g + u
            b = u % 2
            nb = (u + 1) % 2
            _wait_gather(b)
            _wait_e(c, b)
            _compute(b)
            _scatter(b, u)
            _wait_idx(c + 1, nb, (u + 1) % 4)

            @pl.when(c >= 1)
            def _():
                _wait_scatter(nb, (u + 3) % 4)

            _issue_gather(nb)

            @pl.when(c + 2 < NCHUNK)
            def _():
                _front(c + 2, b, (u + 2) % 4)

    _wait_gather(0)
    _wait_e(NCHUNK - 1, 0)
    _compute(0)
    _scatter(0, 0)
    _wait_scatter(1, 3)
    _wait_scatter(0, 0)
    plsc.subcore_barrier()
    pltpu.sync_copy(aggr_s.at[pl.ds(sid * ROWS_PT, ROWS_PT)],
                    out_hbm.at[cid, pl.ds(sid * ROWS_PT, ROWS_PT)])


def _edge_mlp_body(ea_ref, w0, b0, o0):
    a = ea_ref[...]
    o0[...] = jnp.dot(a, w0[...], preferred_element_type=jnp.float32) + b0[...]


_EBLK = 8000


def _edge_mlp(edge_attr, w, b):
    grid = (N_EDGES // _EBLK,)
    espec = pl.BlockSpec((_EBLK, EDGE_DIM), lambda i: (i, 0))
    wspec = pl.BlockSpec((EDGE_DIM, HID), lambda i: (0, 0))
    bspec = pl.BlockSpec((1, HID), lambda i: (0, 0))
    ospec = pl.BlockSpec((_EBLK, HID), lambda i: (i, 0))
    oshape = jax.ShapeDtypeStruct((N_EDGES, HID), jnp.float32)
    return pl.pallas_call(
        _edge_mlp_body,
        grid=grid,
        in_specs=[espec, wspec, bspec],
        out_specs=ospec,
        out_shape=oshape,
    )(edge_attr, w, b)


def _node_body(h_ref, p_ref, s_ref, w1_ref, b1_ref, g1_ref, bt1_ref,
               w2_ref, b2_ref, gn_ref, bn_ref, o_ref, *, residual):
    h = h_ref[...]
    a = s_ref[0, 0] * h + p_ref[0, :N_NODES] + p_ref[1, :N_NODES]
    t = jnp.dot(a, w1_ref[...], preferred_element_type=jnp.float32) + b1_ref[...]
    m = jnp.mean(t, axis=0, keepdims=True)
    v = jnp.mean((t - m) * (t - m), axis=0, keepdims=True)
    t = g1_ref[...] * (t - m) * lax.rsqrt(v + 1e-5) + bt1_ref[...]
    t = jnp.maximum(t, 0.0)
    u = jnp.dot(t, w2_ref[...], preferred_element_type=jnp.float32) + b2_ref[...]
    m2 = jnp.mean(u, axis=0, keepdims=True)
    v2 = jnp.mean((u - m2) * (u - m2), axis=0, keepdims=True)
    u = gn_ref[...] * (u - m2) * lax.rsqrt(v2 + 1e-5) + bn_ref[...]
    u = jnp.maximum(u, 0.0)
    o_ref[...] = h + 0.3 * u if residual else u


def _node_update(h, partials, p, residual):
    s = (1.0 + p['eps']).reshape(1, 1).astype(jnp.float32)
    body = functools.partial(_node_body, residual=residual)
    return pl.pallas_call(
        body,
        out_shape=jax.ShapeDtypeStruct((N_NODES, HID), jnp.float32),
    )(h, partials, s,
      p['W1'].T, p['b1'].reshape(1, -1), p['g1'].reshape(1, -1),
      p['bt1'].reshape(1, -1),
      p['W2'].T, p['b2'].reshape(1, -1), p['gn'].reshape(1, -1),
      p['bn'].reshape(1, -1))


def kernel(x, edge_index, edge_attr, params):
    src = edge_index[0].astype(jnp.int32)
    dst = edge_index[1].astype(jnp.int32)
    e_all = [_edge_mlp(edge_attr, params[i]['We'].T,
                       params[i]['be'].reshape(1, -1)) for i in range(3)]
    h = x
    for i in range(3):
        partials = _mp_sc(h, e_all[i], src, dst)
        h = _node_update(h, partials, params[i], residual=(i == 1))
    return h
